# baseline (speedup 1.0000x reference)
import numpy as np
import jax
import jax.numpy as jnp
from jax.experimental import pallas as pl

_PARENT = [-1, 0, 0, 0, 0, 1, 1, 2, 3, 4, 4, 5, 5, 6, 7, 8, 9, 10,
           11, 12, 13, 14, 14, 14, 15, 15, 16, 17]
_N = 28

_CHILD = [[] for _ in range(_N)]
for _c, _p in enumerate(_PARENT):
    if _p >= 0:
        _CHILD[_p].append(_c)

_deg = np.ones(_N, np.float64)
for _c, _p in enumerate(_PARENT):
    if _p >= 0:
        _deg[_c] += 1.0
        _deg[_p] += 1.0
_dinv = 1.0 / np.sqrt(_deg)
_A = np.zeros((_N, _N), np.float64)
for _i in range(_N):
    _A[_i, _i] = _dinv[_i] * _dinv[_i]
for _c, _p in enumerate(_PARENT):
    if _p >= 0:
        _A[_p, _c] = _dinv[_p] * _dinv[_c]
        _A[_c, _p] = _dinv[_c] * _dinv[_p]
_R = [float(v) for v in _A.sum(1)]
_NBR = [[j for j in range(_N) if _A[i, j] != 0.0] for i in range(_N)]
_DI = [float(v) for v in _dinv]
_DI2 = [float(v * v) for v in _dinv]
_L1C = [float(_dinv[i] * _A[i].sum()) for i in range(_N)]

_ANC = np.zeros((_N, _N), np.float32)
for _i in range(1, _N):
    _n = _i
    while _n != 0:
        _ANC[_i, _n] = 1.0
        _n = _PARENT[_n]

_GROUPS = []
for _p in range(_N):
    _ch = _CHILD[_p]
    if len(_ch) > 1:
        assert _ch == list(range(_ch[0], _ch[0] + len(_ch)))
        _GROUPS.append((_ch[0], _ch[0] + len(_ch)))
_MASK = np.zeros((_N, 1), np.float32)
for _a, _b in _GROUPS:
    _MASK[_a:_b] = 1.0


def _body(x_ref, W0T_ref, b0s_ref, W1T_ref, b1s_ref, W2T_ref, b2s_ref,
          W3T_ref, b3s_ref, w4_ref, b4_ref, anc_ref, mask_ref,
          pp_ref, lg_ref):
    f32 = jnp.float32
    bB = x_ref.shape[0]
    xb = x_ref[...]
    y0T = jax.lax.dot_general(W0T_ref[...], xb, (((1,), (1,)), ((), ())),
                              preferred_element_type=f32)
    b0s = b0s_ref[...]
    h = jnp.concatenate(
        [jax.nn.relu(_L1C[i] * y0T + b0s[:, i:i + 1]) for i in range(_N)],
        axis=1)

    for WT_ref, bs_ref in ((W1T_ref, b1s_ref), (W2T_ref, b2s_ref),
                           (W3T_ref, b3s_ref)):
        p = jnp.dot(WT_ref[...], h, preferred_element_type=f32)
        bs = bs_ref[...]
        h = jnp.concatenate([
            jax.nn.relu(_DI2[i] * sum(p[:, j * bB:(j + 1) * bB]
                                      for j in _NBR[i]) + bs[:, i:i + 1])
            for i in range(_N)], axis=1)

    w4 = w4_ref[...]
    z = jnp.sum(h * w4, axis=0, keepdims=True)
    b4 = b4_ref[0, 0]
    lg = jnp.concatenate([
        _DI[i] * sum(z[:, j * bB:(j + 1) * bB] for j in _NBR[i])
        for i in range(_N)], axis=0) + b4

    lses = []
    for a, bnd in _GROUPS:
        seg = lg[a:bnd]
        m = jnp.max(seg, axis=0, keepdims=True)
        lse = m + jnp.log(jnp.sum(jnp.exp(seg - m), axis=0, keepdims=True))
        lses.append(jnp.broadcast_to(lse, (bnd - a, bB)))
    zrow = jnp.zeros((1, bB), f32)
    lse_node = jnp.concatenate([
        zrow,
        lses[0],
        lses[1],
        jnp.broadcast_to(zrow, (2, bB)),
        lses[2],
        lses[3],
        jnp.broadcast_to(zrow, (8, bB)),
        lses[4],
        lses[5],
        jnp.broadcast_to(zrow, (2, bB)),
    ], axis=0)
    s = mask_ref[...] * lg - lse_node
    logp = jnp.dot(anc_ref[...], s, preferred_element_type=f32)
    pp_ref[...] = jnp.exp(logp).T
    lg_ref[...] = lg.T


def kernel(x, W0, b0, W1, b1, W2, b2, W3, b3, W4, b4):
    B = x.shape[0]
    bB = 512
    while B % bB:
        bB //= 2
    drow = jnp.asarray(np.asarray(_DI, np.float32).reshape(1, _N))
    args = (x, W0.T, b0.reshape(-1, 1) * drow, W1.T, b1.reshape(-1, 1) * drow,
            W2.T, b2.reshape(-1, 1) * drow, W3.T, b3.reshape(-1, 1) * drow,
            W4.reshape(-1, 1), b4.reshape(1, 1),
            jnp.asarray(_ANC), jnp.asarray(_MASK))
    in_specs = [pl.BlockSpec((bB, x.shape[1]), lambda i: (i, 0))]
    for a in args[1:]:
        in_specs.append(pl.BlockSpec(a.shape, lambda i: (0, 0)))
    out_specs = [pl.BlockSpec((bB, _N), lambda i: (i, 0))] * 2
    out_shape = [jax.ShapeDtypeStruct((B, _N), x.dtype)] * 2
    pp, lg = pl.pallas_call(
        _body, grid=(B // bB,), in_specs=in_specs,
        out_specs=out_specs, out_shape=out_shape)(*args)
    return pp, lg


# speedup vs baseline: 1.1519x; 1.1519x over previous
import numpy as np
import jax
import jax.numpy as jnp
from jax.experimental import pallas as pl

_PARENT = [-1, 0, 0, 0, 0, 1, 1, 2, 3, 4, 4, 5, 5, 6, 7, 8, 9, 10,
           11, 12, 13, 14, 14, 14, 15, 15, 16, 17]
_N = 28

_CHILD = [[] for _ in range(_N)]
for _c, _p in enumerate(_PARENT):
    if _p >= 0:
        _CHILD[_p].append(_c)

_deg = np.ones(_N, np.float64)
for _c, _p in enumerate(_PARENT):
    if _p >= 0:
        _deg[_c] += 1.0
        _deg[_p] += 1.0
_dinv = 1.0 / np.sqrt(_deg)
_A = np.zeros((_N, _N), np.float64)
for _i in range(_N):
    _A[_i, _i] = _dinv[_i] * _dinv[_i]
for _c, _p in enumerate(_PARENT):
    if _p >= 0:
        _A[_p, _c] = _dinv[_p] * _dinv[_c]
        _A[_c, _p] = _dinv[_c] * _dinv[_p]
_R = [float(v) for v in _A.sum(1)]
_ATERMS = [[(j, float(_A[i, j])) for j in range(_N) if _A[i, j] != 0.0]
           for i in range(_N)]

_ANC = np.zeros((_N, _N), np.float32)
for _i in range(1, _N):
    _n = _i
    while _n != 0:
        _ANC[_i, _n] = 1.0
        _n = _PARENT[_n]

_GROUPS = []
for _p in range(_N):
    _ch = _CHILD[_p]
    if len(_ch) > 1:
        assert _ch == list(range(_ch[0], _ch[0] + len(_ch)))
        _GROUPS.append((_ch[0], _ch[0] + len(_ch)))
_MASK = np.zeros((_N, 1), np.float32)
for _a, _b in _GROUPS:
    _MASK[_a:_b] = 1.0


def _body(x_ref, W0T_ref, b0_ref, W1T_ref, b1_ref, W2T_ref, b2_ref,
          W3T_ref, b3_ref, w4_ref, b4_ref, anc_ref, mask_ref,
          pp_ref, lg_ref):
    f32 = jnp.float32
    bB = x_ref.shape[0]
    xb = x_ref[...]
    y0T = jax.lax.dot_general(W0T_ref[...], xb, (((1,), (1,)), ((), ())),
                              preferred_element_type=f32)
    b0 = b0_ref[...]
    h = jnp.concatenate(
        [jax.nn.relu(_R[i] * y0T + b0) for i in range(_N)], axis=1)

    for WT_ref, b_ref in ((W1T_ref, b1_ref), (W2T_ref, b2_ref),
                          (W3T_ref, b3_ref)):
        g = jnp.dot(WT_ref[...], h, preferred_element_type=f32)
        b = b_ref[...]
        h = jnp.concatenate([
            jax.nn.relu(sum(c * g[:, j * bB:(j + 1) * bB]
                            for j, c in _ATERMS[i]) + b)
            for i in range(_N)], axis=1)

    w4 = w4_ref[...]
    z = jnp.sum(h * w4, axis=0, keepdims=True)
    b4 = b4_ref[0, 0]
    lg = jnp.concatenate([
        sum(c * z[:, j * bB:(j + 1) * bB] for j, c in _ATERMS[i])
        for i in range(_N)], axis=0) + b4

    lses = []
    for a, bnd in _GROUPS:
        seg = lg[a:bnd]
        m = jnp.max(seg, axis=0, keepdims=True)
        lse = m + jnp.log(jnp.sum(jnp.exp(seg - m), axis=0, keepdims=True))
        lses.append(jnp.broadcast_to(lse, (bnd - a, bB)))
    zrow = jnp.zeros((1, bB), f32)
    lse_node = jnp.concatenate([
        zrow,
        lses[0],
        lses[1],
        jnp.broadcast_to(zrow, (2, bB)),
        lses[2],
        lses[3],
        jnp.broadcast_to(zrow, (8, bB)),
        lses[4],
        lses[5],
        jnp.broadcast_to(zrow, (2, bB)),
    ], axis=0)
    s = mask_ref[...] * lg - lse_node
    logp = jnp.dot(anc_ref[...], s, preferred_element_type=f32)
    pp_ref[...] = jnp.exp(logp).T
    lg_ref[...] = lg.T


def kernel(x, W0, b0, W1, b1, W2, b2, W3, b3, W4, b4):
    B = x.shape[0]
    bB = 512
    while B % bB:
        bB //= 2
    args = (x, W0.T, b0.reshape(-1, 1), W1.T, b1.reshape(-1, 1),
            W2.T, b2.reshape(-1, 1), W3.T, b3.reshape(-1, 1),
            W4.reshape(-1, 1), b4.reshape(1, 1),
            jnp.asarray(_ANC), jnp.asarray(_MASK))
    in_specs = [pl.BlockSpec((bB, x.shape[1]), lambda i: (i, 0))]
    for a in args[1:]:
        in_specs.append(pl.BlockSpec(a.shape, lambda i: (0, 0)))
    out_specs = [pl.BlockSpec((bB, _N), lambda i: (i, 0))] * 2
    out_shape = [jax.ShapeDtypeStruct((B, _N), x.dtype)] * 2
    pp, lg = pl.pallas_call(
        _body, grid=(B // bB,), in_specs=in_specs,
        out_specs=out_specs, out_shape=out_shape)(*args)
    return pp, lg


# speedup vs baseline: 1.2516x; 1.0866x over previous
import numpy as np
import jax
import jax.numpy as jnp
from jax.experimental import pallas as pl

_PARENT = [-1, 0, 0, 0, 0, 1, 1, 2, 3, 4, 4, 5, 5, 6, 7, 8, 9, 10,
           11, 12, 13, 14, 14, 14, 15, 15, 16, 17]
_N = 28

_CHILD = [[] for _ in range(_N)]
for _c, _p in enumerate(_PARENT):
    if _p >= 0:
        _CHILD[_p].append(_c)

_deg = np.ones(_N, np.float64)
for _c, _p in enumerate(_PARENT):
    if _p >= 0:
        _deg[_c] += 1.0
        _deg[_p] += 1.0
_dinv = 1.0 / np.sqrt(_deg)
_A = np.zeros((_N, _N), np.float64)
for _i in range(_N):
    _A[_i, _i] = _dinv[_i] * _dinv[_i]
for _c, _p in enumerate(_PARENT):
    if _p >= 0:
        _A[_p, _c] = _dinv[_p] * _dinv[_c]
        _A[_c, _p] = _dinv[_c] * _dinv[_p]
_R = [float(v) for v in _A.sum(1)]
_ATERMS = [[(j, float(_A[i, j])) for j in range(_N) if _A[i, j] != 0.0]
           for i in range(_N)]

_ANC = np.zeros((_N, _N), np.float32)
for _i in range(1, _N):
    _n = _i
    while _n != 0:
        _ANC[_i, _n] = 1.0
        _n = _PARENT[_n]

_GROUPS = []
for _p in range(_N):
    _ch = _CHILD[_p]
    if len(_ch) > 1:
        assert _ch == list(range(_ch[0], _ch[0] + len(_ch)))
        _GROUPS.append((_ch[0], _ch[0] + len(_ch)))
_MASK = np.zeros((_N, 1), np.float32)
for _a, _b in _GROUPS:
    _MASK[_a:_b] = 1.0


def _body(x_ref, W0T_ref, b0_ref, W1T_ref, b1_ref, W2T_ref, b2_ref,
          W3T_ref, b3_ref, w4_ref, b4_ref, anc_ref, mask_ref,
          pp_ref, lg_ref):
    f32 = jnp.float32
    bB = x_ref.shape[0]
    xb = x_ref[...]
    y0T = jax.lax.dot_general(W0T_ref[...], xb, (((1,), (1,)), ((), ())),
                              preferred_element_type=f32)
    b0 = b0_ref[...]
    h = jnp.concatenate(
        [jax.nn.relu(_R[i] * y0T + b0) for i in range(_N)], axis=1)

    for WT_ref, b_ref in ((W1T_ref, b1_ref), (W2T_ref, b2_ref),
                          (W3T_ref, b3_ref)):
        g = jnp.dot(WT_ref[...], h, preferred_element_type=f32)
        b = b_ref[...]
        h = jnp.concatenate([
            jax.nn.relu(sum(c * g[:, j * bB:(j + 1) * bB]
                            for j, c in _ATERMS[i]) + b)
            for i in range(_N)], axis=1)

    w4 = w4_ref[...]
    z = jnp.sum(h * w4, axis=0, keepdims=True)
    b4 = b4_ref[0, 0]
    lg = jnp.concatenate([
        sum(c * z[:, j * bB:(j + 1) * bB] for j, c in _ATERMS[i])
        for i in range(_N)], axis=0) + b4

    lses = []
    for a, bnd in _GROUPS:
        seg = lg[a:bnd]
        m = jnp.max(seg, axis=0, keepdims=True)
        lse = m + jnp.log(jnp.sum(jnp.exp(seg - m), axis=0, keepdims=True))
        lses.append(jnp.broadcast_to(lse, (bnd - a, bB)))
    zrow = jnp.zeros((1, bB), f32)
    lse_node = jnp.concatenate([
        zrow,
        lses[0],
        lses[1],
        jnp.broadcast_to(zrow, (2, bB)),
        lses[2],
        lses[3],
        jnp.broadcast_to(zrow, (8, bB)),
        lses[4],
        lses[5],
        jnp.broadcast_to(zrow, (2, bB)),
    ], axis=0)
    s = mask_ref[...] * lg - lse_node
    logp = jnp.dot(anc_ref[...], s, preferred_element_type=f32)
    pp_ref[...] = jnp.exp(logp).T
    lg_ref[...] = lg.T


def kernel(x, W0, b0, W1, b1, W2, b2, W3, b3, W4, b4):
    B = x.shape[0]
    bB = 1024
    while B % bB:
        bB //= 2
    args = (x, W0.T, b0.reshape(-1, 1), W1.T, b1.reshape(-1, 1),
            W2.T, b2.reshape(-1, 1), W3.T, b3.reshape(-1, 1),
            W4.reshape(-1, 1), b4.reshape(1, 1),
            jnp.asarray(_ANC), jnp.asarray(_MASK))
    in_specs = [pl.BlockSpec((bB, x.shape[1]), lambda i: (i, 0))]
    for a in args[1:]:
        in_specs.append(pl.BlockSpec(a.shape, lambda i: (0, 0)))
    out_specs = [pl.BlockSpec((bB, _N), lambda i: (i, 0))] * 2
    out_shape = [jax.ShapeDtypeStruct((B, _N), x.dtype)] * 2
    pp, lg = pl.pallas_call(
        _body, grid=(B // bB,), in_specs=in_specs,
        out_specs=out_specs, out_shape=out_shape)(*args)
    return pp, lg
